# baseline (speedup 1.0000x reference)
import functools

import jax
import jax.numpy as jnp
from jax import lax
from jax.experimental import pallas as pl
from jax.experimental.pallas import tpu as pltpu
from jax.experimental.pallas import tpu_sc as plsc

_NUM_CORES = 2
_NUM_SUBCORES = 16
_LANES = 16
_IDX_CHUNK = 128


@functools.lru_cache(maxsize=None)
def _build_sc_kernel(B, D, n_workers, b_per_w, n_chunks):
    mesh = plsc.VectorSubcoreMesh(
        core_axis_name="c",
        subcore_axis_name="s",
        num_cores=_NUM_CORES,
        num_subcores=_NUM_SUBCORES,
    )
    n_groups = _IDX_CHUNK // _LANES
    w_len = 2 * D + 1

    @functools.partial(
        pl.kernel,
        out_type=jax.ShapeDtypeStruct((B,), jnp.float32),
        mesh=mesh,
        compiler_params=pltpu.CompilerParams(
            needs_layout_passes=False, use_tc_tiling_on_sc=False),
        scratch_types=[
            pltpu.VMEM((n_chunks, _IDX_CHUNK), jnp.int32),
            pltpu.VMEM((n_chunks, _IDX_CHUNK), jnp.int32),
            pltpu.VMEM((b_per_w, D), jnp.float32),
            pltpu.VMEM((b_per_w, D), jnp.float32),
            pltpu.VMEM((((w_len + 15) // 16) * 16,), jnp.float32),
            pltpu.VMEM((b_per_w,), jnp.float32),
        ]
        + [pltpu.SemaphoreType.DMA] * (2 * n_chunks),
    )
    def sc_kernel(uid_hbm, mid_hbm, utab_hbm, mtab_hbm, wb_hbm, out_hbm,
                  uid_v, mid_v, urows, mrows, wv, out_v, *sems):
        wid = lax.axis_index("s") * _NUM_CORES + lax.axis_index("c")
        base = wid * b_per_w

        pltpu.sync_copy(uid_hbm.at[wid], uid_v)
        pltpu.sync_copy(mid_hbm.at[wid], mid_v)
        pltpu.sync_copy(wb_hbm, wv)

        copies = []
        for c in range(n_chunks):
            dst = pl.ds(c * _IDX_CHUNK, _IDX_CHUNK)
            copies.append((
                pltpu.async_copy(utab_hbm.at[uid_v.at[c]],
                                 urows.at[dst], sems[2 * c]),
                pltpu.async_copy(mtab_hbm.at[mid_v.at[c]],
                                 mrows.at[dst], sems[2 * c + 1]),
            ))

        w_vecs = [wv[pl.ds(k * _LANES, _LANES)]
                  for k in range((2 * D) // _LANES + 1)]
        bias = w_vecs[(2 * D) // _LANES][(2 * D) % _LANES]
        lane = lax.broadcasted_iota(jnp.int32, (_LANES,), 0)

        for c in range(n_chunks):
            cu, cm = copies[c]
            cu.wait()
            cm.wait()

            def group_body(g, _, c=c):
                r0 = c * _IDX_CHUNK + g * _LANES
                rows = r0 + lane
                accs = [jnp.full((_LANES,), bias, jnp.float32)]
                accs += [jnp.zeros((_LANES,), jnp.float32)] * 3
                for j in range(D):
                    col = jnp.full((_LANES,), j, jnp.int32)
                    u = plsc.load_gather(urows, [rows, col])
                    wj = w_vecs[j // _LANES][j % _LANES]
                    accs[j % 4] = accs[j % 4] + u * wj
                for j in range(D):
                    col = jnp.full((_LANES,), j, jnp.int32)
                    m = plsc.load_gather(mrows, [rows, col])
                    wj = w_vecs[(D + j) // _LANES][j % _LANES]
                    accs[j % 4] = accs[j % 4] + m * wj
                acc = (accs[0] + accs[1]) + (accs[2] + accs[3])
                out_v[pl.ds(r0, _LANES)] = 5.0 / (1.0 + jnp.exp(-acc))
                return 0

            lax.fori_loop(0, n_groups, group_body, 0)

        pltpu.sync_copy(out_v, out_hbm.at[pl.ds(base, b_per_w)])

    return sc_kernel


def kernel(user_ids, movie_ids, user_table, movie_table, fc_w, fc_b):
    B = user_ids.shape[0]
    D = user_table.shape[1]
    n_workers = _NUM_CORES * _NUM_SUBCORES
    b_per_w = B // n_workers
    n_chunks = b_per_w // _IDX_CHUNK

    uid3 = user_ids.astype(jnp.int32).reshape(n_workers, n_chunks, _IDX_CHUNK)
    mid3 = movie_ids.astype(jnp.int32).reshape(n_workers, n_chunks, _IDX_CHUNK)
    w_len = 2 * D + 1
    pad = ((w_len + 15) // 16) * 16 - w_len
    wb = jnp.concatenate([
        fc_w.reshape(-1).astype(jnp.float32),
        fc_b.reshape(-1).astype(jnp.float32),
        jnp.zeros((pad,), jnp.float32),
    ])

    sc = _build_sc_kernel(B, D, n_workers, b_per_w, n_chunks)
    out = sc(uid3, mid3, user_table, movie_table, wb)
    return out.reshape(B, 1)


# speedup vs baseline: 1.0014x; 1.0014x over previous
import functools

import jax
import jax.numpy as jnp
from jax import lax
from jax.experimental import pallas as pl
from jax.experimental.pallas import tpu as pltpu
from jax.experimental.pallas import tpu_sc as plsc

_NUM_CORES = 2
_NUM_SUBCORES = 16
_LANES = 16
_IDX_CHUNK = 128


@functools.lru_cache(maxsize=None)
def _build_sc_kernel(B, D, b_per_w, n_chunks):
    mesh = plsc.VectorSubcoreMesh(
        core_axis_name="c",
        subcore_axis_name="s",
        num_cores=_NUM_CORES,
        num_subcores=_NUM_SUBCORES,
    )
    n_groups = _IDX_CHUNK // _LANES
    w_len = 2 * D + 1
    w_pad = ((w_len + 15) // 16) * 16
    D2 = 2 * D

    @functools.partial(
        pl.kernel,
        out_type=jax.ShapeDtypeStruct((B,), jnp.float32),
        mesh=mesh,
        compiler_params=pltpu.CompilerParams(
            needs_layout_passes=False, use_tc_tiling_on_sc=False),
        scratch_types=[
            pltpu.VMEM((n_chunks, _IDX_CHUNK), jnp.int32),
            pltpu.VMEM((n_chunks, _IDX_CHUNK), jnp.int32),
            pltpu.VMEM((n_chunks, _IDX_CHUNK), jnp.int32),
            pltpu.VMEM((n_chunks, _IDX_CHUNK), jnp.int32),
            pltpu.VMEM((_IDX_CHUNK, D2), jnp.float32),
            pltpu.VMEM((_IDX_CHUNK, D2), jnp.float32),
            pltpu.VMEM((_IDX_CHUNK, D2), jnp.float32),
            pltpu.VMEM((_IDX_CHUNK, D2), jnp.float32),
            pltpu.VMEM((w_pad,), jnp.float32),
            pltpu.VMEM((b_per_w,), jnp.float32),
        ]
        + [pltpu.SemaphoreType.DMA] * 4,
    )
    def sc_kernel(uid_hbm, mid_hbm, utab_hbm, mtab_hbm, wb_hbm, out_hbm,
                  uid_v, mid_v, uix_v, mix_v, ub0, ub1, mb0, mb1, wv,
                  out_v, su0, su1, sm0, sm1):
        wid = lax.axis_index("s") * _NUM_CORES + lax.axis_index("c")
        base = wid * b_per_w

        pltpu.sync_copy(uid_hbm.at[wid], uid_v)
        pltpu.sync_copy(mid_hbm.at[wid], mid_v)
        pltpu.sync_copy(wb_hbm, wv)

        for c in range(n_chunks):
            for k in range(_IDX_CHUNK // _LANES):
                s = pl.ds(k * _LANES, _LANES)
                uix_v[c, s] = lax.shift_right_logical(uid_v[c, s], 1)
                mix_v[c, s] = lax.shift_right_logical(mid_v[c, s], 1)

        ubufs, mbufs = (ub0, ub1), (mb0, mb1)
        usems, msems = (su0, su1), (sm0, sm1)

        def fire(c):
            b = c % 2
            return (
                pltpu.async_copy(utab_hbm.at[uix_v.at[c]], ubufs[b],
                                 usems[b]),
                pltpu.async_copy(mtab_hbm.at[mix_v.at[c]], mbufs[b],
                                 msems[b]),
            )

        w_vecs = [wv[pl.ds(k * _LANES, _LANES)]
                  for k in range(D2 // _LANES + 1)]
        bias = w_vecs[D2 // _LANES][D2 % _LANES]
        lane = lax.broadcasted_iota(jnp.int32, (_LANES,), 0)

        pend = fire(0)
        for c in range(n_chunks):
            cu, cm = pend
            cu.wait()
            cm.wait()
            if c + 1 < n_chunks:
                pend = fire(c + 1)
            ub, mb = ubufs[c % 2], mbufs[c % 2]

            def group_body(g, _, c=c, ub=ub, mb=mb):
                rows = g * _LANES + lane
                s = pl.ds(g * _LANES, _LANES)
                ucol0 = lax.shift_left(
                    lax.bitwise_and(uid_v[c, s], 1), 6)
                mcol0 = lax.shift_left(
                    lax.bitwise_and(mid_v[c, s], 1), 6)
                accs = [jnp.full((_LANES,), bias, jnp.float32)]
                accs += [jnp.zeros((_LANES,), jnp.float32)] * 3
                for j in range(D):
                    u = plsc.load_gather(ub, [rows, ucol0 + j])
                    wj = w_vecs[j // _LANES][j % _LANES]
                    accs[j % 4] = accs[j % 4] + u * wj
                for j in range(D):
                    m = plsc.load_gather(mb, [rows, mcol0 + j])
                    wj = w_vecs[(D + j) // _LANES][j % _LANES]
                    accs[j % 4] = accs[j % 4] + m * wj
                acc = (accs[0] + accs[1]) + (accs[2] + accs[3])
                out_v[pl.ds(c * _IDX_CHUNK + g * _LANES, _LANES)] = (
                    5.0 / (1.0 + jnp.exp(-acc)))
                return 0

            lax.fori_loop(0, n_groups, group_body, 0)

        pltpu.sync_copy(out_v, out_hbm.at[pl.ds(base, b_per_w)])

    return sc_kernel


def kernel(user_ids, movie_ids, user_table, movie_table, fc_w, fc_b):
    B = user_ids.shape[0]
    D = user_table.shape[1]
    n_workers = _NUM_CORES * _NUM_SUBCORES
    b_per_w = B // n_workers
    n_chunks = b_per_w // _IDX_CHUNK

    uid3 = user_ids.astype(jnp.int32).reshape(n_workers, n_chunks, _IDX_CHUNK)
    mid3 = movie_ids.astype(jnp.int32).reshape(n_workers, n_chunks, _IDX_CHUNK)
    ut2 = user_table.reshape(user_table.shape[0] // 2, 2 * D)
    mt2 = movie_table.reshape(movie_table.shape[0] // 2, 2 * D)
    w_len = 2 * D + 1
    pad = ((w_len + 15) // 16) * 16 - w_len
    wb = jnp.concatenate([
        fc_w.reshape(-1).astype(jnp.float32),
        fc_b.reshape(-1).astype(jnp.float32),
        jnp.zeros((pad,), jnp.float32),
    ])

    sc = _build_sc_kernel(B, D, b_per_w, n_chunks)
    out = sc(uid3, mid3, ut2, mt2, wb)
    return out.reshape(B, 1)


# speedup vs baseline: 3.0589x; 3.0547x over previous
import functools

import jax
import jax.numpy as jnp
from jax import lax
from jax.experimental import pallas as pl
from jax.experimental.pallas import tpu as pltpu
from jax.experimental.pallas import tpu_sc as plsc

_NUM_CORES = 2
_NUM_SUBCORES = 16
_LANES = 16
_IDX_CHUNK = 128
_MV_BLK = 4096


@functools.lru_cache(maxsize=None)
def _build_matvec(D, N):
    grid = (N + _MV_BLK - 1) // _MV_BLK

    def body(w_ref, t_ref, z_ref):
        z_ref[...] = jnp.dot(w_ref[...], t_ref[...],
                             preferred_element_type=jnp.float32)

    return pl.pallas_call(
        body,
        grid=(grid,),
        in_specs=[
            pl.BlockSpec((D,), lambda i: (0,)),
            pl.BlockSpec((D, _MV_BLK), lambda i: (0, i)),
        ],
        out_specs=pl.BlockSpec((_MV_BLK,), lambda i: (i,)),
        out_shape=jax.ShapeDtypeStruct((N,), jnp.float32),
    )


@functools.lru_cache(maxsize=None)
def _build_sc_gather(B, b_per_w, n_chunks):
    mesh = plsc.VectorSubcoreMesh(
        core_axis_name="c",
        subcore_axis_name="s",
        num_cores=_NUM_CORES,
        num_subcores=_NUM_SUBCORES,
    )

    @functools.partial(
        pl.kernel,
        out_type=jax.ShapeDtypeStruct((B,), jnp.float32),
        mesh=mesh,
        compiler_params=pltpu.CompilerParams(
            needs_layout_passes=False, use_tc_tiling_on_sc=False),
        scratch_types=[
            pltpu.VMEM((b_per_w,), jnp.int32),
            pltpu.VMEM((b_per_w,), jnp.int32),
            pltpu.VMEM((b_per_w,), jnp.float32),
            pltpu.VMEM((b_per_w,), jnp.float32),
            pltpu.VMEM((_LANES,), jnp.float32),
            pltpu.VMEM((b_per_w,), jnp.float32),
            pltpu.SemaphoreType.DMA,
            pltpu.SemaphoreType.DMA,
        ],
    )
    def sc_kernel(uid_hbm, mid_hbm, zu_hbm, zm_hbm, wb_hbm, out_hbm,
                  uid_v, mid_v, zu_v, zm_v, wv, out_v, su, sm):
        wid = lax.axis_index("s") * _NUM_CORES + lax.axis_index("c")
        base = wid * b_per_w

        pltpu.sync_copy(uid_hbm.at[pl.ds(base, b_per_w)], uid_v)
        pltpu.sync_copy(mid_hbm.at[pl.ds(base, b_per_w)], mid_v)
        pltpu.sync_copy(wb_hbm, wv)

        copies = []
        for c in range(n_chunks):
            s = pl.ds(c * _IDX_CHUNK, _IDX_CHUNK)
            copies.append((
                pltpu.async_copy(zu_hbm.at[uid_v.at[s]], zu_v.at[s], su),
                pltpu.async_copy(zm_hbm.at[mid_v.at[s]], zm_v.at[s], sm),
            ))

        bias = wv[pl.ds(0, _LANES)][0]
        for cu, cm in copies:
            cu.wait()
            cm.wait()

        def group_body(g, _):
            s = pl.ds(g * _LANES, _LANES)
            acc = zu_v[s] + zm_v[s] + bias
            out_v[s] = 5.0 / (1.0 + jnp.exp(-acc))
            return 0

        lax.fori_loop(0, b_per_w // _LANES, group_body, 0)

        pltpu.sync_copy(out_v, out_hbm.at[pl.ds(base, b_per_w)])

    return sc_kernel


def kernel(user_ids, movie_ids, user_table, movie_table, fc_w, fc_b):
    B = user_ids.shape[0]
    D = user_table.shape[1]
    n_workers = _NUM_CORES * _NUM_SUBCORES
    b_per_w = B // n_workers
    n_chunks = b_per_w // _IDX_CHUNK

    w = fc_w.reshape(-1).astype(jnp.float32)
    w_u, w_m = w[:D], w[D:]
    wb = jnp.concatenate([fc_b.reshape(-1).astype(jnp.float32),
                          jnp.zeros((_LANES - 1,), jnp.float32)])

    zu = _build_matvec(D, user_table.shape[0])(w_u, user_table.T)
    zm = _build_matvec(D, movie_table.shape[0])(w_m, movie_table.T)

    sc = _build_sc_gather(B, b_per_w, n_chunks)
    out = sc(user_ids.astype(jnp.int32), movie_ids.astype(jnp.int32),
             zu, zm, wb)
    return out.reshape(B, 1)


# speedup vs baseline: 6.3386x; 2.0722x over previous
import functools

import jax
import jax.numpy as jnp
from jax import lax
from jax.experimental import pallas as pl
from jax.experimental.pallas import tpu as pltpu
from jax.experimental.pallas import tpu_sc as plsc

_NUM_CORES = 2
_NUM_SUBCORES = 16
_LANES = 16
_IDX_CHUNK = 128
_MV_BLK = 32768


@functools.lru_cache(maxsize=None)
def _build_matvec(D, N):
    grid = (N + _MV_BLK - 1) // _MV_BLK

    def body(w_ref, t_ref, z_ref):
        z_ref[...] = jnp.dot(w_ref[...], t_ref[...],
                             preferred_element_type=jnp.float32)

    return pl.pallas_call(
        body,
        grid=(grid,),
        in_specs=[
            pl.BlockSpec((D,), lambda i: (0,)),
            pl.BlockSpec((D, _MV_BLK), lambda i: (0, i)),
        ],
        out_specs=pl.BlockSpec((_MV_BLK,), lambda i: (i,)),
        out_shape=jax.ShapeDtypeStruct((N,), jnp.float32),
    )


@functools.lru_cache(maxsize=None)
def _build_sc_gather(B, b_per_w, n_chunks):
    mesh = plsc.VectorSubcoreMesh(
        core_axis_name="c",
        subcore_axis_name="s",
        num_cores=_NUM_CORES,
        num_subcores=_NUM_SUBCORES,
    )

    @functools.partial(
        pl.kernel,
        out_type=jax.ShapeDtypeStruct((B,), jnp.float32),
        mesh=mesh,
        compiler_params=pltpu.CompilerParams(
            needs_layout_passes=False, use_tc_tiling_on_sc=False),
        scratch_types=[
            pltpu.VMEM((b_per_w,), jnp.int32),
            pltpu.VMEM((b_per_w,), jnp.int32),
            pltpu.VMEM((b_per_w,), jnp.float32),
            pltpu.VMEM((b_per_w,), jnp.float32),
            pltpu.VMEM((_LANES,), jnp.float32),
            pltpu.VMEM((b_per_w,), jnp.float32),
            pltpu.SemaphoreType.DMA,
            pltpu.SemaphoreType.DMA,
        ],
    )
    def sc_kernel(uid_hbm, mid_hbm, zu_hbm, zm_hbm, wb_hbm, out_hbm,
                  uid_v, mid_v, zu_v, zm_v, wv, out_v, su, sm):
        wid = lax.axis_index("s") * _NUM_CORES + lax.axis_index("c")
        base = wid * b_per_w

        pltpu.sync_copy(uid_hbm.at[pl.ds(base, b_per_w)], uid_v)
        pltpu.sync_copy(mid_hbm.at[pl.ds(base, b_per_w)], mid_v)
        pltpu.sync_copy(wb_hbm, wv)

        copies = []
        for c in range(n_chunks):
            s = pl.ds(c * _IDX_CHUNK, _IDX_CHUNK)
            copies.append((
                pltpu.async_copy(zu_hbm.at[uid_v.at[s]], zu_v.at[s], su),
                pltpu.async_copy(zm_hbm.at[mid_v.at[s]], zm_v.at[s], sm),
            ))

        bias = wv[pl.ds(0, _LANES)][0]
        for cu, cm in copies:
            cu.wait()
            cm.wait()

        def group_body(g, _):
            s = pl.ds(g * _LANES, _LANES)
            acc = zu_v[s] + zm_v[s] + bias
            out_v[s] = 5.0 / (1.0 + jnp.exp(-acc))
            return 0

        lax.fori_loop(0, b_per_w // _LANES, group_body, 0)

        pltpu.sync_copy(out_v, out_hbm.at[pl.ds(base, b_per_w)])

    return sc_kernel


def kernel(user_ids, movie_ids, user_table, movie_table, fc_w, fc_b):
    B = user_ids.shape[0]
    D = user_table.shape[1]
    n_workers = _NUM_CORES * _NUM_SUBCORES
    b_per_w = B // n_workers
    n_chunks = b_per_w // _IDX_CHUNK

    w = fc_w.reshape(-1).astype(jnp.float32)
    w_u, w_m = w[:D], w[D:]
    wb = jnp.concatenate([fc_b.reshape(-1).astype(jnp.float32),
                          jnp.zeros((_LANES - 1,), jnp.float32)])

    zu = _build_matvec(D, user_table.shape[0])(w_u, user_table.T)
    zm = _build_matvec(D, movie_table.shape[0])(w_m, movie_table.T)

    sc = _build_sc_gather(B, b_per_w, n_chunks)
    out = sc(user_ids.astype(jnp.int32), movie_ids.astype(jnp.int32),
             zu, zm, wb)
    return out.reshape(B, 1)
